# baseline (speedup 1.0000x reference)
import functools
import math

import jax
import jax.numpy as jnp
from jax import lax
from jax.experimental import pallas as pl
from jax.experimental.pallas import tpu as pltpu
from jax.experimental.pallas import tpu_sc as plsc

EMB_SIZE = 64
SCALE = math.sqrt(EMB_SIZE)

NUM_CORES = 2
NUM_SUBCORES = 16
NUM_WORKERS = NUM_CORES * NUM_SUBCORES
LANES = 16
TOK_PAD = 128
GSEQ = 56

CB = 4


def _emb_body(n_batch, seq, n_chunks, tok_hbm, table_hbm, out_hbm, idx_v,
              rows_v, flat_v, sem0, sem1):
  sems = (sem0, sem1)
  bpw = n_batch // NUM_WORKERS
  wid = lax.axis_index("s") * NUM_CORES + lax.axis_index("c")
  b0 = wid * bpw

  pltpu.sync_copy(tok_hbm.at[pl.ds(b0, bpw), pl.ds(0, 64)], idx_v)

  def gather_descs(k, b):
    return [
        pltpu.make_async_copy(
            table_hbm.at[idx_v.at[k * CB + j, pl.ds(0, GSEQ)]],
            rows_v.at[b, j],
            sems[b],
        )
        for j in range(CB)
    ]

  def issue(k, b):
    for d in gather_descs(k, b):
      d.start()

  issue(0, 0)

  @pl.loop(0, n_chunks // 2)
  def _(k2):
    for b in range(2):
      k = k2 * 2 + b

      @pl.when(k + 1 < n_chunks)
      def _():
        issue(k + 1, 1 - b)

      for d in gather_descs(k, b):
        d.wait()

      for j in range(CB):

        @pl.loop(0, seq)
        def _(i):
          for jj in range(EMB_SIZE // LANES):
            flat_v[b, pl.ds(j * seq * EMB_SIZE + i * EMB_SIZE + jj * LANES,
                            LANES)] = (
                rows_v[b, j, i, pl.ds(jj * LANES, LANES)] * SCALE
            )

      pltpu.sync_copy(
          flat_v.at[b],
          out_hbm.at[pl.ds((b0 + k * CB) * seq * EMB_SIZE,
                           CB * seq * EMB_SIZE)],
      )


@jax.jit
def _emb_call(tok_padded, table):
  n_batch = tok_padded.shape[0]
  seq = 50
  n_chunks = n_batch // NUM_WORKERS // CB
  mesh = plsc.VectorSubcoreMesh(
      core_axis_name="c", subcore_axis_name="s", num_cores=NUM_CORES
  )
  return pl.kernel(
      functools.partial(_emb_body, n_batch, seq, n_chunks),
      out_type=jax.ShapeDtypeStruct((n_batch * seq * EMB_SIZE,), jnp.float32),
      mesh=mesh,
      scratch_types=[
          pltpu.VMEM((n_batch // NUM_WORKERS, 64), jnp.int32),
          pltpu.VMEM((2, CB, GSEQ, EMB_SIZE), jnp.float32),
          pltpu.VMEM((2, CB * seq * EMB_SIZE), jnp.float32),
          pltpu.SemaphoreType.DMA,
          pltpu.SemaphoreType.DMA,
      ],
      compiler_params=pltpu.CompilerParams(use_tc_tiling_on_sc=False),
  )(tok_padded, table)


def kernel(tokens, embedding_weight):
  n_batch, seq = tokens.shape
  assert seq == 50 and n_batch % (NUM_WORKERS * CB * 2) == 0
  tok_padded = jnp.pad(tokens.astype(jnp.int32), ((0, 0), (0, TOK_PAD - seq)))
  out1d = _emb_call(tok_padded, embedding_weight)
  return out1d.reshape(n_batch, seq, EMB_SIZE)


# speedup vs baseline: 2.6885x; 2.6885x over previous
import functools
import math

import jax
import jax.numpy as jnp
from jax import lax
from jax.experimental import pallas as pl
from jax.experimental.pallas import tpu as pltpu
from jax.experimental.pallas import tpu_sc as plsc

EMB_SIZE = 64
SCALE = math.sqrt(EMB_SIZE)

NUM_CORES = 2
NUM_SUBCORES = 16
NUM_WORKERS = NUM_CORES * NUM_SUBCORES
LANES = 16

IDX_ROW = 128
GATHERS_PER_CHUNK = 2
CHUNK = IDX_ROW * GATHERS_PER_CHUNK


def _emb_body(n_tok, n_chunks, tok_hbm, table_hbm, out_hbm, idx_v, rows_v,
              flat_v, sem_i, sem0, sem1):
  sems = (sem0, sem1)
  tpw = n_tok // NUM_WORKERS
  n_rows = tpw // IDX_ROW
  wid = lax.axis_index("s") * NUM_CORES + lax.axis_index("c")
  t0 = wid * tpw

  def stage_desc(q):
    return pltpu.make_async_copy(
        tok_hbm.at[pl.ds(t0 + q * IDX_ROW, IDX_ROW)], idx_v.at[q], sem_i
    )

  @pl.loop(0, n_rows)
  def _(q):
    stage_desc(q).start()

  @pl.loop(0, n_rows)
  def _(q):
    stage_desc(q).wait()

  def gather_descs(k, b):
    return [
        pltpu.make_async_copy(
            table_hbm.at[idx_v.at[k * GATHERS_PER_CHUNK + j]],
            rows_v.at[b, pl.ds(j * IDX_ROW, IDX_ROW), :],
            sems[b],
        )
        for j in range(GATHERS_PER_CHUNK)
    ]

  def issue(k, b):
    for d in gather_descs(k, b):
      d.start()

  issue(0, 0)

  @pl.loop(0, n_chunks // 2)
  def _(k2):
    for b in range(2):
      k = k2 * 2 + b

      @pl.when(k + 1 < n_chunks)
      def _():
        issue(k + 1, 1 - b)

      for d in gather_descs(k, b):
        d.wait()

      @pl.loop(0, CHUNK)
      def _(r):
        for jj in range(EMB_SIZE // LANES):
          flat_v[b, pl.ds(r * EMB_SIZE + jj * LANES, LANES)] = (
              rows_v[b, r, pl.ds(jj * LANES, LANES)] * SCALE
          )

      pltpu.sync_copy(
          flat_v.at[b],
          out_hbm.at[pl.ds((t0 + k * CHUNK) * EMB_SIZE, CHUNK * EMB_SIZE)],
      )


@jax.jit
def _emb_call(tok_flat, table):
  n_tok = tok_flat.shape[0]
  tpw = n_tok // NUM_WORKERS
  n_chunks = tpw // CHUNK
  mesh = plsc.VectorSubcoreMesh(
      core_axis_name="c", subcore_axis_name="s", num_cores=NUM_CORES
  )
  return pl.kernel(
      functools.partial(_emb_body, n_tok, n_chunks),
      out_type=jax.ShapeDtypeStruct((n_tok * EMB_SIZE,), jnp.float32),
      mesh=mesh,
      scratch_types=[
          pltpu.VMEM((tpw // IDX_ROW, IDX_ROW), jnp.int32),
          pltpu.VMEM((2, CHUNK, EMB_SIZE), jnp.float32),
          pltpu.VMEM((2, CHUNK * EMB_SIZE), jnp.float32),
          pltpu.SemaphoreType.DMA,
          pltpu.SemaphoreType.DMA,
          pltpu.SemaphoreType.DMA,
      ],
      compiler_params=pltpu.CompilerParams(use_tc_tiling_on_sc=False),
  )(tok_flat, table)


def kernel(tokens, embedding_weight):
  n_batch, seq = tokens.shape
  n_tok = n_batch * seq
  assert n_tok % (NUM_WORKERS * CHUNK * 2) == 0
  tok_flat = tokens.astype(jnp.int32).reshape(-1)
  out1d = _emb_call(tok_flat, embedding_weight)
  return out1d.reshape(n_batch, seq, EMB_SIZE)
